# Initial kernel scaffold; baseline (speedup 1.0000x reference)
import jax
import jax.numpy as jnp
from jax.experimental import pallas as pl


def kernel(z, W):
    raise NotImplementedError("write your pallas kernel here")


# speedup vs baseline: 1.1990x; 1.1990x over previous
import jax
import jax.numpy as jnp
from jax import lax
from jax.experimental import pallas as pl
from jax.experimental.pallas import tpu as pltpu
from jax.experimental.pallas import tpu_sc as plsc

_N = 16384
_K = 8192
_D = 256
_TN = 512
_TK = 2048
_LANES = 128

_NW = 32
_ROWS_PER_W = _N // _NW
_GC = 128


def _dist_argmin_body(z_ref, z2_ref, w_ref, w2_ref, idx_ref, dsum_ref,
                      accv_ref, acci_ref):
    k = pl.program_id(1)

    @pl.when(k == 0)
    def _init():
        accv_ref[...] = jnp.full((_TN, _LANES), jnp.inf, jnp.float32)
        acci_ref[...] = jnp.zeros((_TN, _LANES), jnp.int32)

    m = lax.dot_general(z_ref[...], w_ref[...], (((1,), (1,)), ((), ())),
                        preferred_element_type=jnp.float32)
    x = z2_ref[...] + w2_ref[...]
    d = x - 2.0 * m
    kbase = k * _TK
    accv = accv_ref[...]
    acci = acci_ref[...]
    lane = lax.broadcasted_iota(jnp.int32, (_TN, _LANES), 1)
    for j in range(_TK // _LANES):
        dj = d[:, j * _LANES:(j + 1) * _LANES]
        kv = lane + (kbase + j * _LANES)
        mask = dj < accv
        accv = jnp.where(mask, dj, accv)
        acci = jnp.where(mask, kv, acci)
    accv_ref[...] = accv
    acci_ref[...] = acci

    @pl.when(k == pl.num_programs(1) - 1)
    def _fin():
        rowmin = jnp.min(accv, axis=1, keepdims=True)
        cand = jnp.where(accv == rowmin, acci, jnp.int32(_K))
        idx_ref[...] = jnp.min(cand, axis=1, keepdims=True)
        s = jnp.sum(rowmin, keepdims=True)

        @pl.when(pl.program_id(0) == 0)
        def _zero():
            dsum_ref[...] = jnp.zeros((1, 1), jnp.float32)

        dsum_ref[...] += s


def _argmin_call(z, z2, w, w2):
    return pl.pallas_call(
        _dist_argmin_body,
        grid=(_N // _TN, _K // _TK),
        in_specs=[
            pl.BlockSpec((_TN, _D), lambda n, k: (n, 0)),
            pl.BlockSpec((_TN, 1), lambda n, k: (n, 0)),
            pl.BlockSpec((_TK, _D), lambda n, k: (k, 0)),
            pl.BlockSpec((1, _TK), lambda n, k: (0, k)),
        ],
        out_specs=[
            pl.BlockSpec((_TN, 1), lambda n, k: (n, 0)),
            pl.BlockSpec((1, 1), lambda n, k: (0, 0)),
        ],
        out_shape=[
            jax.ShapeDtypeStruct((_N, 1), jnp.int32),
            jax.ShapeDtypeStruct((1, 1), jnp.float32),
        ],
        scratch_shapes=[
            pltpu.VMEM((_TN, _LANES), jnp.float32),
            pltpu.VMEM((_TN, _LANES), jnp.int32),
        ],
        compiler_params=pltpu.CompilerParams(
            dimension_semantics=("parallel", "arbitrary")),
    )(z, z2, w, w2)


def _sc_gather_body(w_hbm, idx_hbm, out_hbm, idx_v, rows_v, sem):
    wid = lax.axis_index("s") * 2 + lax.axis_index("c")
    base = wid * _ROWS_PER_W
    for c in range(_ROWS_PER_W // _GC):
        off = base + c * _GC
        pltpu.sync_copy(idx_hbm.at[pl.ds(off, _GC)], idx_v)
        pltpu.async_copy(w_hbm.at[idx_v], rows_v, sem).wait()
        pltpu.sync_copy(rows_v, out_hbm.at[pl.ds(off, _GC)])


def _gather_call(w, idx):
    return pl.kernel(
        _sc_gather_body,
        out_type=jax.ShapeDtypeStruct((_N, _D), jnp.float32),
        mesh=plsc.VectorSubcoreMesh(core_axis_name="c", subcore_axis_name="s"),
        scratch_types=[
            pltpu.VMEM((_GC,), jnp.int32),
            pltpu.VMEM((_GC, _D), jnp.float32),
            pltpu.SemaphoreType.DMA,
        ],
    )(w, idx)


def kernel(z, W):
    z2 = jnp.sum(z ** 2, axis=1, keepdims=True)
    w2 = jnp.sum(W ** 2, axis=1)[None, :]
    idx2d, dsum = _argmin_call(z, z2, W, w2)
    quantized = _gather_call(W, idx2d.reshape(_N))
    x = dsum[0, 0] / jnp.float32(_N * _D)
    loss = x + jnp.float32(0.25) * x
    return quantized, loss, idx2d
